# baseline (speedup 1.0000x reference)
import functools

import jax
import jax.numpy as jnp
from jax import lax
from jax.experimental import pallas as pl
from jax.experimental.pallas import tpu as pltpu
from jax.experimental.pallas import tpu_sc as plsc

_NC = 2
_NS = 16
_NW = _NC * _NS
_LANES = 16
_CH = 8


def _coef_body(u_ref, a_ref, w_ref, c_ref):
    u = u_ref[...]
    t = jnp.dot(a_ref[...], u, precision=lax.Precision.HIGHEST,
                preferred_element_type=jnp.float32)
    s = t * w_ref[...]
    c_ref[...] = lax.dot_general(
        s, u, (((1,), (1,)), ((), ())), precision=lax.Precision.HIGHEST,
        preferred_element_type=jnp.float32)


@functools.partial(jax.jit, static_argnums=(0,))
def _coefficients(K, weight, avgweight, U):
    u_pad = jnp.zeros((128, 128), jnp.float32).at[:K, :K].set(U)
    a_row = jnp.zeros((8, 128), jnp.float32).at[0, :K].set(avgweight[:, 0])
    w_row = jnp.zeros((8, 128), jnp.float32).at[0, :K].set(weight[:, 0])
    return pl.pallas_call(
        _coef_body,
        out_shape=jax.ShapeDtypeStruct((8, 128), jnp.float32),
    )(u_pad, a_row, w_row)


def _make_sc_kernel(b_pad, d, s_slots, ch0, ch1, nbuf):
    mesh = plsc.VectorSubcoreMesh(core_axis_name="c", subcore_axis_name="s")
    grp = _CH * s_slots
    chmax = max(ch0, ch1)

    scratch = [pltpu.VMEM((chmax, grp), jnp.int32)]
    scratch += [pltpu.VMEM((grp, d), jnp.float32) for _ in range(nbuf)]
    scratch += [pltpu.VMEM((_CH, d), jnp.float32) for _ in range(nbuf)]
    scratch += [pltpu.VMEM((s_slots, _LANES), jnp.float32)]
    scratch += [pltpu.SemaphoreType.DMA for _ in range(2 * nbuf)]

    @functools.partial(
        pl.kernel,
        mesh=mesh,
        out_type=jax.ShapeDtypeStruct((b_pad, d), jnp.float32),
        scratch_types=scratch,
    )
    def sc_k(idx_hbm, table_hbm, cb_hbm, out_hbm, *sc):
        idx_v = sc[0]
        rows = sc[1:1 + nbuf]
        outs = sc[1 + nbuf:1 + 2 * nbuf]
        cb_v = sc[1 + 2 * nbuf]
        sgs = sc[2 + 2 * nbuf:2 + 3 * nbuf]
        sos = sc[2 + 3 * nbuf:2 + 4 * nbuf]
        cid = lax.axis_index("c")
        sid = lax.axis_index("s")
        pltpu.sync_copy(cb_hbm, cb_v)
        nv = d // _LANES

        def compute(rv, ov):
            zero = jnp.zeros((_LANES,), jnp.float32)
            for r0 in range(0, _CH, 2):
                def s_body(s, accs):
                    cs = cb_v[s, :]
                    return tuple(
                        accs[i] + cs * rv[(r0 + i // nv) * s_slots + s,
                                          pl.ds((i % nv) * _LANES, _LANES)]
                        for i in range(2 * nv))
                accs = lax.fori_loop(0, s_slots, s_body, (zero,) * (2 * nv))
                for i in range(2 * nv):
                    ov[r0 + i // nv, pl.ds((i % nv) * _LANES, _LANES)] = \
                        accs[i]

        def run(gbase, nch):
            pltpu.sync_copy(idx_hbm.at[pl.ds(gbase, nch)],
                            idx_v.at[pl.ds(0, nch)])
            base = gbase * _CH
            for b in range(nbuf):
                pltpu.async_copy(table_hbm.at[idx_v.at[b]], rows[b], sgs[b])

            def grp_body(jg, carry):
                for b in range(nbuf):
                    j = nbuf * jg + b
                    pltpu.make_async_copy(
                        table_hbm.at[idx_v.at[j]], rows[b], sgs[b]).wait()

                    @pl.when(jg > 0)
                    def _drain_prev_write():
                        pltpu.make_async_copy(
                            outs[b], out_hbm.at[pl.ds(0, _CH)],
                            sos[b]).wait()

                    compute(rows[b], outs[b])
                    pltpu.async_copy(
                        outs[b], out_hbm.at[pl.ds(base + j * _CH, _CH)],
                        sos[b])

                    @pl.when(j + nbuf < nch)
                    def _start_next_gather():
                        pltpu.async_copy(
                            table_hbm.at[idx_v.at[j + nbuf]], rows[b],
                            sgs[b])

                return carry

            lax.fori_loop(0, nch // nbuf, grp_body, 0)
            for b in range(nbuf):
                pltpu.make_async_copy(
                    outs[b], out_hbm.at[pl.ds(0, _CH)], sos[b]).wait()

        if ch0 == ch1:
            run((sid * _NC + cid) * ch0, ch0)
        else:
            @pl.when(cid == 0)
            def _core0():
                run(sid * ch0, ch0)

            @pl.when(cid == 1)
            def _core1():
                run(_NS * ch0 + sid * ch1, ch1)

    return sc_k


def kernel(feat_table, neighbor_idx, weight, avgweight, U):
    B, S = neighbor_idx.shape
    D = feat_table.shape[1]
    K = U.shape[0]

    c_row = _coefficients(K, weight, avgweight, U)
    cb = jnp.broadcast_to(c_row[0, 1:1 + S].reshape(S, 1), (S, _LANES))

    nbuf = 2
    step = _NS * _CH * 8
    b_pad = ((B + step - 1) // step) * step
    n_chunks = b_pad // (_NS * _CH)
    ch0 = 72
    ch1 = n_chunks - ch0
    idx = neighbor_idx.astype(jnp.int32)
    idx_p = jnp.zeros((b_pad, S), jnp.int32).at[:B].set(idx)
    idx_r = idx_p.reshape(b_pad // _CH, _CH * S)

    out_p = _make_sc_kernel(b_pad, D, S, ch0, ch1, nbuf)(idx_r, feat_table,
                                                         cb)
    return out_p[:B]


# speedup vs baseline: 1.0126x; 1.0126x over previous
import functools

import jax
import jax.numpy as jnp
from jax import lax
from jax.experimental import pallas as pl
from jax.experimental.pallas import tpu as pltpu
from jax.experimental.pallas import tpu_sc as plsc

_NC = 2
_NS = 16
_NW = _NC * _NS
_LANES = 16
_CH = 8


def _coef_body(u_ref, a_ref, w_ref, c_ref):
    u = u_ref[...]
    t = jnp.dot(a_ref[...], u, precision=lax.Precision.HIGHEST,
                preferred_element_type=jnp.float32)
    s = t * w_ref[...]
    c_ref[...] = lax.dot_general(
        s, u, (((1,), (1,)), ((), ())), precision=lax.Precision.HIGHEST,
        preferred_element_type=jnp.float32)


@functools.partial(jax.jit, static_argnums=(0,))
def _coefficients(K, weight, avgweight, U):
    u_pad = jnp.zeros((128, 128), jnp.float32).at[:K, :K].set(U)
    a_row = jnp.zeros((8, 128), jnp.float32).at[0, :K].set(avgweight[:, 0])
    w_row = jnp.zeros((8, 128), jnp.float32).at[0, :K].set(weight[:, 0])
    return pl.pallas_call(
        _coef_body,
        out_shape=jax.ShapeDtypeStruct((8, 128), jnp.float32),
    )(u_pad, a_row, w_row)


def _make_sc_kernel(b_pad, d, s_slots, ch0, ch1, nbuf):
    mesh = plsc.VectorSubcoreMesh(core_axis_name="c", subcore_axis_name="s")
    grp = _CH * s_slots
    chmax = max(ch0, ch1)

    scratch = [pltpu.VMEM((chmax, grp), jnp.int32)]
    scratch += [pltpu.VMEM((grp, d), jnp.float32) for _ in range(nbuf)]
    scratch += [pltpu.VMEM((_CH, d), jnp.float32) for _ in range(nbuf)]
    scratch += [pltpu.VMEM((s_slots, _LANES), jnp.float32)]
    scratch += [pltpu.SemaphoreType.DMA for _ in range(2 * nbuf)]

    @functools.partial(
        pl.kernel,
        mesh=mesh,
        out_type=jax.ShapeDtypeStruct((b_pad, d), jnp.float32),
        scratch_types=scratch,
    )
    def sc_k(idx_hbm, table_hbm, cb_hbm, out_hbm, *sc):
        idx_v = sc[0]
        rows = sc[1:1 + nbuf]
        outs = sc[1 + nbuf:1 + 2 * nbuf]
        cb_v = sc[1 + 2 * nbuf]
        sgs = sc[2 + 2 * nbuf:2 + 3 * nbuf]
        sos = sc[2 + 3 * nbuf:2 + 4 * nbuf]
        cid = lax.axis_index("c")
        sid = lax.axis_index("s")
        pltpu.sync_copy(cb_hbm, cb_v)
        nv = d // _LANES

        def compute(rv, ov):
            zero = jnp.zeros((_LANES,), jnp.float32)
            for r0 in range(0, _CH, 2):
                def s_body(s, accs):
                    cs = cb_v[s, :]
                    return tuple(
                        accs[i] + cs * rv[(r0 + i // nv) * s_slots + s,
                                          pl.ds((i % nv) * _LANES, _LANES)]
                        for i in range(2 * nv))
                accs = lax.fori_loop(0, s_slots, s_body, (zero,) * (2 * nv))
                for i in range(2 * nv):
                    ov[r0 + i // nv, pl.ds((i % nv) * _LANES, _LANES)] = \
                        accs[i]

        def run(gbase, nch):
            pltpu.sync_copy(idx_hbm.at[pl.ds(gbase, nch)],
                            idx_v.at[pl.ds(0, nch)])
            base = gbase * _CH
            for b in range(nbuf):
                pltpu.async_copy(table_hbm.at[idx_v.at[b]], rows[b], sgs[b])

            def grp_body(jg, carry):
                for b in range(nbuf):
                    j = nbuf * jg + b
                    pltpu.make_async_copy(
                        table_hbm.at[idx_v.at[j]], rows[b], sgs[b]).wait()

                    @pl.when(jg > 0)
                    def _drain_prev_write():
                        pltpu.make_async_copy(
                            outs[b], out_hbm.at[pl.ds(0, _CH)],
                            sos[b]).wait()

                    compute(rows[b], outs[b])
                    pltpu.async_copy(
                        outs[b], out_hbm.at[pl.ds(base + j * _CH, _CH)],
                        sos[b])

                    @pl.when(j + nbuf < nch)
                    def _start_next_gather():
                        pltpu.async_copy(
                            table_hbm.at[idx_v.at[j + nbuf]], rows[b],
                            sgs[b])

                return carry

            lax.fori_loop(0, nch // nbuf, grp_body, 0)
            for b in range(nbuf):
                pltpu.make_async_copy(
                    outs[b], out_hbm.at[pl.ds(0, _CH)], sos[b]).wait()

        if ch0 == ch1:
            run((sid * _NC + cid) * ch0, ch0)
        else:
            @pl.when(cid == 0)
            def _core0():
                run(sid * ch0, ch0)

            @pl.when(cid == 1)
            def _core1():
                run(_NS * ch0 + sid * ch1, ch1)

    return sc_k


def kernel(feat_table, neighbor_idx, weight, avgweight, U):
    B, S = neighbor_idx.shape
    D = feat_table.shape[1]
    K = U.shape[0]

    c_row = _coefficients(K, weight, avgweight, U)
    cb = jnp.broadcast_to(c_row[0, 1:1 + S].reshape(S, 1), (S, _LANES))

    nbuf = 2
    step = _NS * _CH * 8
    b_pad = ((B + step - 1) // step) * step
    n_chunks = b_pad // (_NS * _CH)
    ch0 = n_chunks // 2
    ch1 = n_chunks - ch0
    idx = neighbor_idx.astype(jnp.int32)
    idx_p = jnp.zeros((b_pad, S), jnp.int32).at[:B].set(idx)
    idx_r = idx_p.reshape(b_pad // _CH, _CH * S)

    out_p = _make_sc_kernel(b_pad, D, S, ch0, ch1, nbuf)(idx_r, feat_table,
                                                         cb)
    return out_p[:B]


# speedup vs baseline: 2.7368x; 2.7026x over previous
import functools

import jax
import jax.numpy as jnp
from jax import lax
from jax.experimental import pallas as pl
from jax.experimental.pallas import tpu as pltpu
from jax.experimental.pallas import tpu_sc as plsc

_NC = 2
_NS = 16
_NW = _NC * _NS
_LANES = 16
_CH = 8


def _coef_body(u_ref, a_ref, w_ref, c_ref):
    u = u_ref[...]
    t = jnp.dot(a_ref[...], u, precision=lax.Precision.HIGHEST,
                preferred_element_type=jnp.float32)
    s = t * w_ref[...]
    c_ref[...] = lax.dot_general(
        s, u, (((1,), (1,)), ((), ())), precision=lax.Precision.HIGHEST,
        preferred_element_type=jnp.float32)


@functools.partial(jax.jit, static_argnums=(0,))
def _coefficients(K, weight, avgweight, U):
    u_pad = jnp.zeros((128, 128), jnp.float32).at[:K, :K].set(U)
    a_row = jnp.zeros((8, 128), jnp.float32).at[0, :K].set(avgweight[:, 0])
    w_row = jnp.zeros((8, 128), jnp.float32).at[0, :K].set(weight[:, 0])
    return pl.pallas_call(
        _coef_body,
        out_shape=jax.ShapeDtypeStruct((8, 128), jnp.float32),
    )(u_pad, a_row, w_row)


def _make_sc_kernel(b_pad, d, s_slots, n_chunks, nbuf):
    mesh = plsc.VectorSubcoreMesh(core_axis_name="c", subcore_axis_name="s")
    grp = _CH * s_slots

    scratch = [pltpu.VMEM((n_chunks, grp), jnp.int32)]
    scratch += [pltpu.VMEM((grp, d), jnp.float32) for _ in range(nbuf)]
    scratch += [pltpu.VMEM((_CH, d), jnp.float32) for _ in range(nbuf)]
    scratch += [pltpu.VMEM((s_slots, _LANES), jnp.float32)]
    scratch += [pltpu.SemaphoreType.DMA for _ in range(2 * nbuf)]

    @functools.partial(
        pl.kernel,
        mesh=mesh,
        out_type=jax.ShapeDtypeStruct((b_pad, d), jnp.float32),
        scratch_types=scratch,
    )
    def sc_k(idx_hbm, table_hbm, cb_hbm, out_hbm, *sc):
        idx_v = sc[0]
        rows = sc[1:1 + nbuf]
        outs = sc[1 + nbuf:1 + 2 * nbuf]
        cb_v = sc[1 + 2 * nbuf]
        sgs = sc[2 + 2 * nbuf:2 + 3 * nbuf]
        sos = sc[2 + 3 * nbuf:2 + 4 * nbuf]
        wid = lax.axis_index("s") * _NC + lax.axis_index("c")
        pltpu.sync_copy(cb_hbm, cb_v)
        pltpu.sync_copy(idx_hbm.at[wid], idx_v)
        nv = d // _LANES

        def compute(rv, ov):
            zero = jnp.zeros((_LANES,), jnp.float32)
            for r0 in range(0, _CH, 2):
                def s_body(s, accs):
                    cs = cb_v[s, :]
                    return tuple(
                        accs[i] + cs * rv[(r0 + i // nv) * s_slots + s,
                                          pl.ds((i % nv) * _LANES, _LANES)]
                        for i in range(2 * nv))
                accs = lax.fori_loop(0, s_slots, s_body, (zero,) * (2 * nv))
                for i in range(2 * nv):
                    ov[r0 + i // nv, pl.ds((i % nv) * _LANES, _LANES)] = \
                        accs[i]

        for b in range(nbuf):
            pltpu.async_copy(table_hbm.at[idx_v.at[b]], rows[b], sgs[b])

        def grp_body(jg, carry):
            for b in range(nbuf):
                j = nbuf * jg + b
                pltpu.make_async_copy(
                    table_hbm.at[idx_v.at[j]], rows[b], sgs[b]).wait()

                @pl.when(jg > 0)
                def _drain_prev_write():
                    pltpu.make_async_copy(
                        outs[b], out_hbm.at[pl.ds(0, _CH)], sos[b]).wait()

                compute(rows[b], outs[b])
                pltpu.async_copy(
                    outs[b],
                    out_hbm.at[pl.ds((wid + _NW * j) * _CH, _CH)], sos[b])

                @pl.when(j + nbuf < n_chunks)
                def _start_next_gather():
                    pltpu.async_copy(
                        table_hbm.at[idx_v.at[j + nbuf]], rows[b], sgs[b])

            return carry

        lax.fori_loop(0, n_chunks // nbuf, grp_body, 0)
        for b in range(nbuf):
            pltpu.make_async_copy(
                outs[b], out_hbm.at[pl.ds(0, _CH)], sos[b]).wait()

    return sc_k


def kernel(feat_table, neighbor_idx, weight, avgweight, U):
    B, S = neighbor_idx.shape
    D = feat_table.shape[1]
    K = U.shape[0]

    c_row = _coefficients(K, weight, avgweight, U)
    cb = jnp.broadcast_to(c_row[0, 1:1 + S].reshape(S, 1), (S, _LANES))

    nbuf = 2
    step = _NW * _CH * nbuf
    b_pad = ((B + step - 1) // step) * step
    n_chunks = b_pad // (_NW * _CH)
    idx = neighbor_idx.astype(jnp.int32)
    idx_p = jnp.zeros((b_pad, S), jnp.int32).at[:B].set(idx)
    idx_r = jnp.transpose(
        idx_p.reshape(n_chunks, _NW, _CH * S), (1, 0, 2))

    out_p = _make_sc_kernel(b_pad, D, S, n_chunks, nbuf)(idx_r, feat_table,
                                                         cb)
    return out_p[:B]
